# baseline (speedup 1.0000x reference)
import functools

import jax
import jax.numpy as jnp
from jax import lax
from jax.experimental import pallas as pl
from jax.experimental.pallas import tpu as pltpu
from jax.experimental.pallas import tpu_sc as plsc

EMBED_DIM = 64
CHUNK = 128


def kernel(x, weight):
    B, H = x.shape
    V = weight.shape[0]
    N = B * H
    info = plsc.get_sparse_core_info()
    NC, NS = info.num_cores, info.num_subcores
    NW = NC * NS
    n_units = N // CHUNK
    u_per_w = n_units // NW
    BBLK = B // CHUNK
    assert n_units * CHUNK == N and u_per_w * NW == n_units
    assert u_per_w % 2 == 0

    xt = x.T.reshape(H * BBLK, CHUNK)
    xq = (xt >> 1).reshape(NW, u_per_w, CHUNK)
    sel = ((xt & 1) << 6).reshape(NW, u_per_w, CHUNK)
    w2 = weight.reshape(V // 2, 2 * EMBED_DIM)

    mesh = plsc.VectorSubcoreMesh(core_axis_name="c", subcore_axis_name="s")

    @functools.partial(
        pl.kernel,
        mesh=mesh,
        out_type=jax.ShapeDtypeStruct((H, 8, BBLK, 8, CHUNK), jnp.float32),
        scratch_types=[
            pltpu.VMEM((u_per_w, CHUNK), jnp.int32),
            pltpu.VMEM((u_per_w, CHUNK), jnp.int32),
            pltpu.VMEM((2, CHUNK, 2 * EMBED_DIM), jnp.float32),
            pltpu.VMEM((2, 8, 8, CHUNK), jnp.float32),
            pltpu.SemaphoreType.DMA,
            pltpu.SemaphoreType.DMA,
            pltpu.SemaphoreType.DMA,
            pltpu.SemaphoreType.DMA,
        ],
        compiler_params=pltpu.CompilerParams(
            use_tc_tiling_on_sc=True, needs_layout_passes=False
        ),
    )
    def run(xq_hbm, sel_hbm, w2_hbm, out_hbm, idx_v, sel_v, buf, tv,
            sg0, sg1, so0, so1):
        wid = lax.axis_index("s") * NC + lax.axis_index("c")
        base_u = wid * u_per_w
        pltpu.sync_copy(xq_hbm.at[wid], idx_v)
        pltpu.sync_copy(sel_hbm.at[wid], sel_v)
        lanes = lax.iota(jnp.int32, 16)
        sg = (sg0, sg1)
        so = (so0, so1)

        def gather_start(u, b):
            pltpu.make_async_copy(w2_hbm.at[idx_v.at[u]], buf.at[b], sg[b]).start()

        def gather_wait(b):
            pltpu.make_async_copy(
                w2_hbm.at[idx_v.at[0]], buf.at[b], sg[b]
            ).wait()

        def out_dst(u):
            f = base_u + u
            h = f // BBLK
            bc = f - h * BBLK
            return out_hbm.at[h, :, bc]

        def out_start(u, b):
            pltpu.make_async_copy(tv.at[b], out_dst(u), so[b]).start()

        def out_wait(b):
            pltpu.make_async_copy(tv.at[b], out_dst(0), so[b]).wait()

        def extract(u, b):
            for g in range(CHUNK // 16):
                rows = lanes + (g * 16)
                cols0 = sel_v[u, pl.ds(g * 16, 16)]
                for d in range(EMBED_DIM):
                    vals = plsc.load_gather(buf.at[b], [rows, cols0 + d])
                    tv[b, d // 8, d % 8, pl.ds(g * 16, 16)] = vals

        gather_start(0, 0)
        gather_start(1, 1)

        def step(j, carry):
            u = j * 2
            for b in range(2):
                gather_wait(b)
                extract(u + b, b)
                gather_next = u + b + 2

                @pl.when(gather_next < u_per_w)
                def _():
                    gather_start(gather_next, b)

                @pl.when(j > 0)
                def _():
                    out_wait(b)

                out_start(u + b, b)
            return carry

        lax.fori_loop(0, u_per_w // 2, step, 0)
        out_wait(0)
        out_wait(1)

    out5 = run(xq, sel, w2)
    return out5.transpose(2, 4, 0, 1, 3).reshape(B, H, EMBED_DIM)


# speedup vs baseline: 1.0486x; 1.0486x over previous
import functools

import jax
import jax.numpy as jnp
from jax import lax
from jax.experimental import pallas as pl
from jax.experimental.pallas import tpu as pltpu
from jax.experimental.pallas import tpu_sc as plsc

EMBED_DIM = 64
CHUNK = 128


def kernel(x, weight):
    B, H = x.shape
    V = weight.shape[0]
    N = B * H
    info = plsc.get_sparse_core_info()
    NC, NS = info.num_cores, info.num_subcores
    NW = NC * NS
    n_units = N // CHUNK
    u_per_w = n_units // NW
    BBLK = B // CHUNK
    assert n_units * CHUNK == N and u_per_w * NW == n_units
    assert u_per_w % 2 == 0

    xt = x.T.reshape(H * BBLK, CHUNK)
    xq = (xt >> 1).reshape(NW, u_per_w, CHUNK)
    sel = ((xt & 1) << 6).reshape(NW, u_per_w, CHUNK)
    w2 = weight.reshape(V // 2, 2 * EMBED_DIM)

    mesh = plsc.VectorSubcoreMesh(core_axis_name="c", subcore_axis_name="s")

    @functools.partial(
        pl.kernel,
        mesh=mesh,
        out_type=jax.ShapeDtypeStruct((H, 8, BBLK, 8, CHUNK), jnp.float32),
        scratch_types=[
            pltpu.VMEM((u_per_w, CHUNK), jnp.int32),
            pltpu.VMEM((u_per_w, CHUNK), jnp.int32),
            pltpu.VMEM((2, CHUNK, 2 * EMBED_DIM), jnp.float32),
            pltpu.VMEM((2, 8, 8, CHUNK), jnp.float32),
            pltpu.SemaphoreType.DMA,
            pltpu.SemaphoreType.DMA,
            pltpu.SemaphoreType.DMA,
            pltpu.SemaphoreType.DMA,
        ],
        compiler_params=pltpu.CompilerParams(
            use_tc_tiling_on_sc=True, needs_layout_passes=False
        ),
    )
    def run(xq_hbm, sel_hbm, w2_hbm, out_hbm, idx_v, sel_v, buf, tv,
            sg0, sg1, so0, so1):
        wid = lax.axis_index("s") * NC + lax.axis_index("c")
        base_u = wid * u_per_w
        pltpu.sync_copy(xq_hbm.at[wid], idx_v)
        pltpu.sync_copy(sel_hbm.at[wid], sel_v)
        lanes = lax.iota(jnp.int32, 16)
        sg = (sg0, sg1)
        so = (so0, so1)

        def gather_start(u, b):
            pltpu.make_async_copy(w2_hbm.at[idx_v.at[u]], buf.at[b], sg[b]).start()

        def gather_wait(b):
            pltpu.make_async_copy(
                w2_hbm.at[idx_v.at[0]], buf.at[b], sg[b]
            ).wait()

        def out_dst(u):
            f = base_u + u
            h = f // BBLK
            bc = f - h * BBLK
            return out_hbm.at[h, :, bc]

        def out_start(u, b):
            pltpu.make_async_copy(tv.at[b], out_dst(u), so[b]).start()

        def out_wait(b):
            pltpu.make_async_copy(tv.at[b], out_dst(0), so[b]).wait()

        def extract(u, b):
            def gstep(g, carry_g):
                g16 = g * 16
                rows = lanes + g16
                cols0 = sel_v[u, pl.ds(g16, 16)]

                def dstep(dd, carry_d):
                    for k in range(8):
                        vals = plsc.load_gather(
                            buf.at[b], [rows, cols0 + (dd * 8 + k)]
                        )
                        tv[b, dd, k, pl.ds(g16, 16)] = vals
                    return carry_d

                lax.fori_loop(0, 8, dstep, 0)
                return carry_g

            lax.fori_loop(0, CHUNK // 16, gstep, 0)

        gather_start(0, 0)
        gather_start(1, 1)

        def step(j, carry):
            u = j * 2
            for b in range(2):
                gather_wait(b)
                extract(u + b, b)
                gather_next = u + b + 2

                @pl.when(gather_next < u_per_w)
                def _():
                    gather_start(gather_next, b)

                @pl.when(j > 0)
                def _():
                    out_wait(b)

                out_start(u + b, b)
            return carry

        lax.fori_loop(0, u_per_w // 2, step, 0)
        out_wait(0)
        out_wait(1)

    out5 = run(xq, sel, w2)
    return out5.transpose(2, 4, 0, 1, 3).reshape(B, H, EMBED_DIM)


# speedup vs baseline: 1.8228x; 1.7382x over previous
import functools

import jax
import jax.numpy as jnp
from jax import lax
from jax.experimental import pallas as pl
from jax.experimental.pallas import tpu as pltpu
from jax.experimental.pallas import tpu_sc as plsc

EMBED_DIM = 64
CHUNK = 128


def kernel(x, weight):
    B, H = x.shape
    N = B * H
    info = plsc.get_sparse_core_info()
    NC, NS = info.num_cores, info.num_subcores
    NW = NC * NS
    n_per_w = N // NW
    n_chunks = n_per_w // CHUNK
    assert n_per_w * NW == N and n_chunks * CHUNK == n_per_w

    x_flat = x.reshape(NW, n_chunks, CHUNK).astype(jnp.int32)
    mesh = plsc.VectorSubcoreMesh(core_axis_name="c", subcore_axis_name="s")

    @functools.partial(
        pl.kernel,
        mesh=mesh,
        out_type=jax.ShapeDtypeStruct((N, 2 * EMBED_DIM), jnp.float32),
        scratch_types=[
            pltpu.VMEM((n_chunks, CHUNK), jnp.int32),
            pltpu.VMEM((CHUNK, EMBED_DIM), jnp.float32),
            pltpu.SemaphoreType.DMA,
        ],
        compiler_params=pltpu.CompilerParams(use_tc_tiling_on_sc=False),
    )
    def run(x_hbm, w_hbm, out_hbm, idx_v, rows_v, sem):
        wid = lax.axis_index("s") * NC + lax.axis_index("c")
        base = wid * n_per_w
        pltpu.sync_copy(x_hbm.at[wid], idx_v)

        def body(j, carry):
            pltpu.async_copy(w_hbm.at[idx_v.at[j]], rows_v, sem).wait()
            pltpu.sync_copy(
                rows_v,
                out_hbm.at[pl.ds(base + j * CHUNK, CHUNK), pl.ds(0, EMBED_DIM)],
            )
            return carry

        lax.fori_loop(0, n_chunks, body, 0)

    out = run(x_flat, weight)
    return out[:, :EMBED_DIM].reshape(B, H, EMBED_DIM)
